# baseline (speedup 1.0000x reference)
import jax
import jax.numpy as jnp
from jax import lax
from jax.experimental import pallas as pl
from jax.experimental.pallas import tpu as pltpu
from jax.experimental.pallas import tpu_sc as plsc

_DEPTH = 1000
_OUT = 64
_BATCH = 16384

_NC = 2
_NS = 16
_OBANDS = 2
_BQ = 16
_ROWS = _OUT // _OBANDS
_BPT = _BATCH // _BQ


def _gather_body(table_hbm, idx_hbm, out_hbm, w_v, idx_v, out_v, sem):
    wid = lax.axis_index("s") * _NC + lax.axis_index("c")
    band = wid // _BQ
    group = lax.rem(wid, _BQ)
    c1 = pltpu.async_copy(table_hbm.at[pl.ds(band * _ROWS, _ROWS)], w_v, sem)
    c2 = pltpu.async_copy(idx_hbm.at[0, pl.ds(group * _BPT, _BPT)], idx_v, sem)
    c1.wait()
    c2.wait()

    @plsc.parallel_loop(0, _BPT, step=16, unroll=4)
    def _(col):
        idx_vec = idx_v[pl.ds(col, 16)]
        for o in range(_ROWS):
            row = jnp.full((16,), o, dtype=jnp.int32)
            out_v[o, pl.ds(col, 16)] = plsc.load_gather(w_v, [row, idx_vec])

    pltpu.sync_copy(
        out_v,
        out_hbm.at[pl.ds(band * _ROWS, _ROWS), pl.ds(group * _BPT, _BPT)],
    )


def kernel(input_batch, W, b):
    idx = input_batch.astype(jnp.int32).reshape(1, _BATCH)
    table = W + b[:, None]

    mesh = plsc.VectorSubcoreMesh(core_axis_name="c", subcore_axis_name="s")
    gather = pl.kernel(
        _gather_body,
        mesh=mesh,
        compiler_params=pltpu.CompilerParams(
            use_tc_tiling_on_sc=True, needs_layout_passes=False
        ),
        out_type=jax.ShapeDtypeStruct((_OUT, _BATCH), jnp.float32),
        scratch_types=[
            pltpu.VMEM((_ROWS, _DEPTH), jnp.float32),
            pltpu.VMEM((_BPT,), jnp.int32),
            pltpu.VMEM((_ROWS, _BPT), jnp.float32),
            pltpu.SemaphoreType.DMA,
        ],
    )
    out_t = gather(table, idx)
    return out_t.T[:, None, :]


# speedup vs baseline: 1.1434x; 1.1434x over previous
import jax
import jax.numpy as jnp
from jax import lax
from jax.experimental import pallas as pl
from jax.experimental.pallas import tpu as pltpu
from jax.experimental.pallas import tpu_sc as plsc

_DEPTH = 1000
_OUT = 64
_BATCH = 16384

_NC = 2
_NS = 16
_OBANDS = 8
_BQ = 4
_ROWS = _OUT // _OBANDS
_BPT = _BATCH // _BQ


def _gather_body(table_hbm, idx_hbm, out_hbm, w_v, idx_v, out_v, sem):
    wid = lax.axis_index("s") * _NC + lax.axis_index("c")
    band = wid // _BQ
    group = lax.rem(wid, _BQ)
    c1 = pltpu.async_copy(table_hbm.at[pl.ds(band * _ROWS, _ROWS)], w_v, sem)
    c2 = pltpu.async_copy(idx_hbm.at[0, pl.ds(group * _BPT, _BPT)], idx_v, sem)
    c1.wait()
    c2.wait()

    @plsc.parallel_loop(0, _BPT, step=16, unroll=8)
    def _(col):
        idx_vec = idx_v[pl.ds(col, 16)]
        for o in range(_ROWS):
            row = jnp.full((16,), o, dtype=jnp.int32)
            out_v[o, pl.ds(col, 16)] = plsc.load_gather(w_v, [row, idx_vec])

    pltpu.sync_copy(
        out_v,
        out_hbm.at[pl.ds(band * _ROWS, _ROWS), pl.ds(group * _BPT, _BPT)],
    )


def kernel(input_batch, W, b):
    idx = input_batch.astype(jnp.int32).reshape(1, _BATCH)
    table = W + b[:, None]

    mesh = plsc.VectorSubcoreMesh(core_axis_name="c", subcore_axis_name="s")
    gather = pl.kernel(
        _gather_body,
        mesh=mesh,
        compiler_params=pltpu.CompilerParams(
            use_tc_tiling_on_sc=True, needs_layout_passes=False
        ),
        out_type=jax.ShapeDtypeStruct((_OUT, _BATCH), jnp.float32),
        scratch_types=[
            pltpu.VMEM((_ROWS, _DEPTH), jnp.float32),
            pltpu.VMEM((_BPT,), jnp.int32),
            pltpu.VMEM((_ROWS, _BPT), jnp.float32),
            pltpu.SemaphoreType.DMA,
        ],
    )
    out_t = gather(table, idx)
    return out_t.T[:, None, :]
